# baseline (speedup 1.0000x reference)
import jax
import jax.numpy as jnp
from jax.experimental import pallas as pl
from jax.experimental.pallas import tpu as pltpu

_BLOCK = 4096


def _fused_kernel_t(pid_ref, iid_ref, cid_ref, tp_ref, ti_ref, tc_ref,
                    w1_ref, b1_ref, w2_ref, b2_ref, out_ref):
    blk = pid_ref.shape[0]
    vp, dp = tp_ref.shape
    vi, di = ti_ref.shape
    vc, dc = tc_ref.shape
    rp = 8 * ((vp + 7) // 8)
    ri = 8 * ((vi + 7) // 8)
    rc = 8 * ((vc + 7) // 8)
    mp = jnp.dot(tp_ref[...], w1_ref[0:dp, :], preferred_element_type=jnp.float32)
    mi = jnp.dot(ti_ref[...], w1_ref[dp:dp + di, :], preferred_element_type=jnp.float32)
    mc = jnp.dot(tc_ref[...], w1_ref[dp + di:dp + di + dc, :], preferred_element_type=jnp.float32)
    pid = pid_ref[...][None, :]
    iid = iid_ref[...][None, :]
    cid = cid_ref[...][None, :]
    ohp = (jax.lax.broadcasted_iota(jnp.int32, (rp, blk), 0) == pid).astype(jnp.float32)
    ohi = (jax.lax.broadcasted_iota(jnp.int32, (ri, blk), 0) == iid).astype(jnp.float32)
    ohc = (jax.lax.broadcasted_iota(jnp.int32, (rc, blk), 0) == cid).astype(jnp.float32)
    oh_all = jnp.concatenate([ohp, ohi, ohc], axis=0)
    d1 = w1_ref.shape[1]
    mt = jnp.concatenate([
        mp.T, jnp.zeros((d1, rp - vp), jnp.float32),
        mi.T, jnp.zeros((d1, ri - vi), jnp.float32),
        mc.T, jnp.zeros((d1, rc - vc), jnp.float32)], axis=1)
    hT = (jnp.dot(mt, oh_all, preferred_element_type=jnp.float32)
          + b1_ref[...][:, None])
    hT = jnp.maximum(hT, 0.0)
    out_ref[...] = (jnp.dot(hT.T, w2_ref[...], preferred_element_type=jnp.float32)
                    + b2_ref[...][None, :])


def kernel(platform_id, industry_id, cta_id, platform_table, industry_table,
           cta_table, W1, b1, W2, b2):
    B = platform_id.shape[0]
    blk = min(_BLOCK, B)
    grid = B // blk
    pid2 = platform_id.astype(jnp.int32)
    iid2 = industry_id.astype(jnp.int32)
    cid2 = cta_id.astype(jnp.int32)
    d_out = W2.shape[1]
    return pl.pallas_call(
        _fused_kernel_t,
        grid=(grid,),
        compiler_params=pltpu.CompilerParams(fuse_transposed_lhs_in_matmul=True),
        in_specs=[
            pl.BlockSpec((blk,), lambda i: (i,)),
            pl.BlockSpec((blk,), lambda i: (i,)),
            pl.BlockSpec((blk,), lambda i: (i,)),
            pl.BlockSpec(platform_table.shape, lambda i: (0, 0)),
            pl.BlockSpec(industry_table.shape, lambda i: (0, 0)),
            pl.BlockSpec(cta_table.shape, lambda i: (0, 0)),
            pl.BlockSpec(W1.shape, lambda i: (0, 0)),
            pl.BlockSpec(b1.shape, lambda i: (0,)),
            pl.BlockSpec(W2.shape, lambda i: (0, 0)),
            pl.BlockSpec(b2.shape, lambda i: (0,)),
        ],
        out_specs=pl.BlockSpec((blk, d_out), lambda i: (i, 0)),
        out_shape=jax.ShapeDtypeStruct((B, d_out), jnp.float32),
    )(pid2, iid2, cid2, platform_table, industry_table, cta_table, W1, b1, W2, b2)


# speedup vs baseline: 1.0009x; 1.0009x over previous
import jax
import jax.numpy as jnp
from jax.experimental import pallas as pl
from jax.experimental.pallas import tpu as pltpu

_BLOCK = 16384


def _fused_kernel_t(pid_ref, iid_ref, cid_ref, tp_ref, ti_ref, tc_ref,
                    w1_ref, b1_ref, w2_ref, b2_ref, out_ref):
    blk = pid_ref.shape[0]
    vp, dp = tp_ref.shape
    vi, di = ti_ref.shape
    vc, dc = tc_ref.shape
    rp = 8 * ((vp + 7) // 8)
    ri = 8 * ((vi + 7) // 8)
    rc = 8 * ((vc + 7) // 8)
    mp = jnp.dot(tp_ref[...], w1_ref[0:dp, :], preferred_element_type=jnp.float32)
    mi = jnp.dot(ti_ref[...], w1_ref[dp:dp + di, :], preferred_element_type=jnp.float32)
    mc = jnp.dot(tc_ref[...], w1_ref[dp + di:dp + di + dc, :], preferred_element_type=jnp.float32)
    pid = pid_ref[...][None, :]
    iid = iid_ref[...][None, :]
    cid = cid_ref[...][None, :]
    ohp = (jax.lax.broadcasted_iota(jnp.int32, (rp, blk), 0) == pid).astype(jnp.float32)
    ohi = (jax.lax.broadcasted_iota(jnp.int32, (ri, blk), 0) == iid).astype(jnp.float32)
    ohc = (jax.lax.broadcasted_iota(jnp.int32, (rc, blk), 0) == cid).astype(jnp.float32)
    oh_all = jnp.concatenate([ohp, ohi, ohc], axis=0)
    d1 = w1_ref.shape[1]
    mt = jnp.concatenate([
        mp.T, jnp.zeros((d1, rp - vp), jnp.float32),
        mi.T, jnp.zeros((d1, ri - vi), jnp.float32),
        mc.T, jnp.zeros((d1, rc - vc), jnp.float32)], axis=1)
    hT = (jnp.dot(mt, oh_all, preferred_element_type=jnp.float32)
          + b1_ref[...][:, None])
    hT = jnp.maximum(hT, 0.0)
    out_ref[...] = (jnp.dot(hT.T, w2_ref[...], preferred_element_type=jnp.float32)
                    + b2_ref[...][None, :])


def kernel(platform_id, industry_id, cta_id, platform_table, industry_table,
           cta_table, W1, b1, W2, b2):
    B = platform_id.shape[0]
    blk = min(_BLOCK, B)
    grid = B // blk
    pid2 = platform_id.astype(jnp.int32)
    iid2 = industry_id.astype(jnp.int32)
    cid2 = cta_id.astype(jnp.int32)
    d_out = W2.shape[1]
    return pl.pallas_call(
        _fused_kernel_t,
        grid=(grid,),
        compiler_params=pltpu.CompilerParams(fuse_transposed_lhs_in_matmul=True),
        in_specs=[
            pl.BlockSpec((blk,), lambda i: (i,)),
            pl.BlockSpec((blk,), lambda i: (i,)),
            pl.BlockSpec((blk,), lambda i: (i,)),
            pl.BlockSpec(platform_table.shape, lambda i: (0, 0)),
            pl.BlockSpec(industry_table.shape, lambda i: (0, 0)),
            pl.BlockSpec(cta_table.shape, lambda i: (0, 0)),
            pl.BlockSpec(W1.shape, lambda i: (0, 0)),
            pl.BlockSpec(b1.shape, lambda i: (0,)),
            pl.BlockSpec(W2.shape, lambda i: (0, 0)),
            pl.BlockSpec(b2.shape, lambda i: (0,)),
        ],
        out_specs=pl.BlockSpec((blk, d_out), lambda i: (i, 0)),
        out_shape=jax.ShapeDtypeStruct((B, d_out), jnp.float32),
    )(pid2, iid2, cid2, platform_table, industry_table, cta_table, W1, b1, W2, b2)


# speedup vs baseline: 1.0375x; 1.0365x over previous
import jax
import jax.numpy as jnp
from jax.experimental import pallas as pl
from jax.experimental.pallas import tpu as pltpu

_BLOCK = 8192


def _fused_kernel_t(pid_ref, iid_ref, cid_ref, tp_ref, ti_ref, tc_ref,
                    w1_ref, b1_ref, w2_ref, b2_ref, out_ref):
    blk = pid_ref.shape[0]
    vp, dp = tp_ref.shape
    vi, di = ti_ref.shape
    vc, dc = tc_ref.shape
    rp = 8 * ((vp + 7) // 8)
    ri = 8 * ((vi + 7) // 8)
    rc = 8 * ((vc + 7) // 8)
    mp = jnp.dot(tp_ref[...], w1_ref[0:dp, :], preferred_element_type=jnp.float32)
    mi = jnp.dot(ti_ref[...], w1_ref[dp:dp + di, :], preferred_element_type=jnp.float32)
    mc = jnp.dot(tc_ref[...], w1_ref[dp + di:dp + di + dc, :], preferred_element_type=jnp.float32)
    pid = pid_ref[...][None, :]
    iid = iid_ref[...][None, :]
    cid = cid_ref[...][None, :]
    ohp = (jax.lax.broadcasted_iota(jnp.int32, (rp, blk), 0) == pid).astype(jnp.float32)
    ohi = (jax.lax.broadcasted_iota(jnp.int32, (ri, blk), 0) == iid).astype(jnp.float32)
    ohc = (jax.lax.broadcasted_iota(jnp.int32, (rc, blk), 0) == cid).astype(jnp.float32)
    oh_all = jnp.concatenate([ohp, ohi, ohc], axis=0)
    d1 = w1_ref.shape[1]
    mt = jnp.concatenate([
        mp.T, jnp.zeros((d1, rp - vp), jnp.float32),
        mi.T, jnp.zeros((d1, ri - vi), jnp.float32),
        mc.T, jnp.zeros((d1, rc - vc), jnp.float32)], axis=1)
    hT = (jnp.dot(mt, oh_all, preferred_element_type=jnp.float32)
          + b1_ref[...][:, None])
    hT = jnp.maximum(hT, 0.0)
    out_ref[...] = (jnp.dot(hT.T, w2_ref[...], preferred_element_type=jnp.float32)
                    + b2_ref[...][None, :])


def kernel(platform_id, industry_id, cta_id, platform_table, industry_table,
           cta_table, W1, b1, W2, b2):
    B = platform_id.shape[0]
    blk = min(_BLOCK, B)
    grid = B // blk
    pid2 = platform_id.astype(jnp.int32)
    iid2 = industry_id.astype(jnp.int32)
    cid2 = cta_id.astype(jnp.int32)
    d_out = W2.shape[1]
    return pl.pallas_call(
        _fused_kernel_t,
        grid=(grid,),
        compiler_params=pltpu.CompilerParams(fuse_transposed_lhs_in_matmul=True),
        in_specs=[
            pl.BlockSpec((blk,), lambda i: (i,)),
            pl.BlockSpec((blk,), lambda i: (i,)),
            pl.BlockSpec((blk,), lambda i: (i,)),
            pl.BlockSpec(platform_table.shape, lambda i: (0, 0)),
            pl.BlockSpec(industry_table.shape, lambda i: (0, 0)),
            pl.BlockSpec(cta_table.shape, lambda i: (0, 0)),
            pl.BlockSpec(W1.shape, lambda i: (0, 0)),
            pl.BlockSpec(b1.shape, lambda i: (0,)),
            pl.BlockSpec(W2.shape, lambda i: (0, 0)),
            pl.BlockSpec(b2.shape, lambda i: (0,)),
        ],
        out_specs=pl.BlockSpec((blk, d_out), lambda i: (i, 0)),
        out_shape=jax.ShapeDtypeStruct((B, d_out), jnp.float32),
    )(pid2, iid2, cid2, platform_table, industry_table, cta_table, W1, b1, W2, b2)
